# Initial kernel scaffold; baseline (speedup 1.0000x reference)
import jax
import jax.numpy as jnp
from jax.experimental import pallas as pl


def kernel(feature_ids_batch, feature_values_batch, W):
    raise NotImplementedError("write your pallas kernel here")


# speedup vs baseline: 1.2037x; 1.2037x over previous
import functools

import jax
import jax.numpy as jnp
from jax import lax
from jax.experimental import pallas as pl
from jax.experimental.pallas import tpu as pltpu
from jax.experimental.pallas import tpu_sc as plsc

BATCH = 16384
ACTIVE = 100

_NC = 2
_NS = 16
_NW = _NC * _NS
_ROWS_PER_W = BATCH // _NW
_CHUNKS = 2
_R = _ROWS_PER_W // _CHUNKS
_CW = _R * ACTIVE


def _sc_kernel(ids_hbm, vals_hbm, w_hbm, out_hbm, idsv, valsv, wv, outv, sem):
    wid = lax.axis_index("s") * _NC + lax.axis_index("c")
    base_row = wid * _ROWS_PER_W
    lane = lax.iota(jnp.int32, 16)
    lane_off = lane * ACTIVE

    for c in range(_CHUNKS):
        off = base_row * ACTIVE + c * _CW
        pltpu.sync_copy(ids_hbm.at[pl.ds(off, _CW)], idsv)
        pltpu.sync_copy(vals_hbm.at[pl.ds(off, _CW)], valsv)
        pltpu.async_copy(w_hbm.at[idsv], wv, sem).wait()

        def group_body(g, _):
            gbase = g * (16 * ACTIVE)

            def a_body(a, acc):
                idx = lane_off + (gbase + a)
                w = plsc.load_gather(wv, [idx])
                v = plsc.load_gather(valsv, [idx])
                return acc + w * v

            acc = lax.fori_loop(0, ACTIVE, a_body, jnp.zeros((16,), jnp.float32))
            y = 1.0 / (1.0 + jnp.exp(-acc))
            outv[pl.ds(c * _R + g * 16, 16)] = y
            return 0

        lax.fori_loop(0, _R // 16, group_body, 0)

    pltpu.sync_copy(outv, out_hbm.at[pl.ds(base_row, _ROWS_PER_W)])


@functools.partial(jax.jit, static_argnames=())
def kernel(feature_ids_batch, feature_values_batch, W):
    ids_flat = feature_ids_batch.reshape(-1).astype(jnp.int32)
    vals_flat = feature_values_batch.reshape(-1)

    mesh = plsc.VectorSubcoreMesh(core_axis_name="c", subcore_axis_name="s")
    out = pl.kernel(
        _sc_kernel,
        mesh=mesh,
        compiler_params=pltpu.CompilerParams(needs_layout_passes=False),
        out_type=jax.ShapeDtypeStruct((BATCH,), jnp.float32),
        scratch_types=[
            pltpu.VMEM((_CW,), jnp.int32),
            pltpu.VMEM((_CW,), jnp.float32),
            pltpu.VMEM((_CW,), jnp.float32),
            pltpu.VMEM((_ROWS_PER_W,), jnp.float32),
            pltpu.SemaphoreType.DMA,
        ],
    )(ids_flat, vals_flat, W)
    return out.reshape(BATCH, 1)
